# Initial kernel scaffold; baseline (speedup 1.0000x reference)
import jax
import jax.numpy as jnp
from jax.experimental import pallas as pl


def kernel(logits):
    raise NotImplementedError("write your pallas kernel here")


# speedup vs baseline: 81.2963x; 81.2963x over previous
import functools

import jax
import jax.numpy as jnp
from jax.experimental import pallas as pl

_TEMP = 0.6
_K = 50
_P = 0.9
_ROW_BLOCK = 8


def _body(x_ref, o_ref):
    x = x_ref[...] / _TEMP
    r = x.shape[0]

    xi = jax.lax.bitcast_convert_type(x, jnp.int32)
    skey = jnp.where(xi < 0, xi ^ jnp.int32(0x7FFFFFFF), xi)

    int_min = jnp.int32(-(2**31))
    zero = jnp.zeros((r, 1), jnp.int32)

    def cnt_ge(c):
        return jnp.sum((skey >= c).astype(jnp.int32), axis=1, keepdims=True)

    t = jnp.where(cnt_ge(zero) >= _K, zero, int_min)

    def b1(i, t):
        bit = jnp.left_shift(jnp.int32(1), jnp.int32(30) - i)
        cand = t + bit
        return jnp.where(cnt_ge(cand) >= _K, cand, t)

    t = jax.lax.fori_loop(0, 31, b1, t)

    m = jnp.max(x, axis=1, keepdims=True)
    ez = jnp.where(skey >= t, jnp.exp(x - m), jnp.float32(0.0))
    s = jnp.sum(ez, axis=1, keepdims=True)
    lim = s * jnp.float32(_P)

    def tail_gt(c):
        return jnp.sum(jnp.where(skey > c, ez, jnp.float32(0.0)), axis=1,
                       keepdims=True)

    u = jnp.where(tail_gt(zero) > lim, zero, int_min)

    def b2(i, u):
        bit = jnp.left_shift(jnp.int32(1), jnp.int32(30) - i)
        cand = u + bit
        return jnp.where(tail_gt(cand) > lim, cand, u)

    u = jax.lax.fori_loop(0, 31, b2, u)

    oe = jnp.where(skey > u, ez, jnp.float32(0.0))
    sf = jnp.sum(oe, axis=1, keepdims=True)
    o_ref[...] = oe / sf


@jax.jit
def kernel(logits):
    n_rows, vocab = logits.shape
    grid = (n_rows // _ROW_BLOCK,)
    return pl.pallas_call(
        _body,
        grid=grid,
        in_specs=[pl.BlockSpec((_ROW_BLOCK, vocab), lambda i: (i, 0))],
        out_specs=pl.BlockSpec((_ROW_BLOCK, vocab), lambda i: (i, 0)),
        out_shape=jax.ShapeDtypeStruct((n_rows, vocab), jnp.float32),
    )(logits)
